# baseline (speedup 1.0000x reference)
import jax
import jax.numpy as jnp
from jax.experimental import pallas as pl
from jax.experimental.pallas import tpu as pltpu

_B = 32
_C = 384
_N = 196
_NTOT = float(_B * _N)
_EPS = 1e-5


def _fused(x_ref, wd_ref, bd_ref, g1_ref, be1_ref, wg_ref, bg_ref,
           wu_ref, bu_ref, g2s_ref, be2s_ref, out_ref,
           y_ref, wc_ref, acc_ref, coef_ref):
    p = pl.program_id(0)
    b = pl.program_id(1)

    @pl.when((p == 0) & (b == 0))
    def _init():
        acc_ref[...] = jnp.zeros((_C, 4), jnp.float32)

    @pl.when(p == 0)
    def _pass1():
        y1 = jnp.dot(wd_ref[...], x_ref[0],
                     preferred_element_type=jnp.float32) + bd_ref[...]
        y_ref[b] = y1
        rs = jnp.sum(y1, axis=1, keepdims=True)
        rq = jnp.sum(y1 * y1, axis=1, keepdims=True)
        acc_ref[:, 0:2] = acc_ref[:, 0:2] + jnp.concatenate([rs, rq], axis=1)

    @pl.when((p == 1) & (b == 0))
    def _mid1():
        mu = acc_ref[:, 0:1] / _NTOT
        var = acc_ref[:, 1:2] / _NTOT - mu * mu
        a1 = g1_ref[...] * jax.lax.rsqrt(var + _EPS)
        coef_ref[:, 0:1] = a1
        coef_ref[:, 1:2] = be1_ref[...] - mu * a1
        wu = wu_ref[...]
        wc_ref[...] = jnp.dot(wu, wg_ref[...],
                              preferred_element_type=jnp.float32)
        coef_ref[:, 2:3] = jnp.dot(wu, bg_ref[...],
                                   preferred_element_type=jnp.float32) + bu_ref[...]

    @pl.when(p == 1)
    def _pass2():
        y2 = y_ref[b] * coef_ref[:, 0:1] + coef_ref[:, 1:2]
        y4 = jnp.dot(wc_ref[...], y2,
                     preferred_element_type=jnp.float32) + coef_ref[:, 2:3]
        y_ref[b] = y4
        rs = jnp.sum(y4, axis=1, keepdims=True)
        rq = jnp.sum(y4 * y4, axis=1, keepdims=True)
        acc_ref[:, 2:4] = acc_ref[:, 2:4] + jnp.concatenate([rs, rq], axis=1)

    @pl.when((p == 2) & (b == 0))
    def _mid2():
        mu = acc_ref[:, 2:3] / _NTOT
        var = acc_ref[:, 3:4] / _NTOT - mu * mu
        a2 = g2s_ref[...] * jax.lax.rsqrt(var + _EPS)
        coef_ref[:, 0:1] = a2
        coef_ref[:, 1:2] = be2s_ref[...] - mu * a2

    @pl.when(p == 2)
    def _pass3():
        out_ref[0] = y_ref[b] * coef_ref[:, 0:1] + coef_ref[:, 1:2]


def kernel(x, batch, W_down, b_down, g1, be1, W_gcn, b_gcn, W_up, b_up,
           g2, be2, rel_pos):
    del rel_pos
    scale = jnp.asarray(batch, jnp.float32) / 8.0
    col = lambda v: v.reshape(_C, 1).astype(jnp.float32)
    xr = x.reshape(_B, _C, _N)

    mat = pl.BlockSpec((_C, _C), lambda p, b: (0, 0))
    vec = pl.BlockSpec((_C, 1), lambda p, b: (0, 0))
    out = pl.pallas_call(
        _fused,
        grid=(3, _B),
        in_specs=[
            pl.BlockSpec((1, _C, _N),
                         lambda p, b: (jnp.where(p == 0, b, _B - 1), 0, 0)),
            mat, vec, vec, vec, mat, vec, mat, vec, vec, vec,
        ],
        out_specs=pl.BlockSpec((1, _C, _N),
                               lambda p, b: (jnp.where(p == 2, b, 0), 0, 0)),
        out_shape=jax.ShapeDtypeStruct((_B, _C, _N), jnp.float32),
        scratch_shapes=[
            pltpu.VMEM((_B, _C, _N), jnp.float32),
            pltpu.VMEM((_C, _C), jnp.float32),
            pltpu.VMEM((_C, 4), jnp.float32),
            pltpu.VMEM((_C, 4), jnp.float32),
        ],
        compiler_params=pltpu.CompilerParams(
            dimension_semantics=("arbitrary", "arbitrary")),
    )(xr, W_down, col(b_down), col(g1), col(be1), W_gcn, col(b_gcn),
      W_up, col(b_up), col(g2 * scale), col(be2 * scale))
    return out.reshape(x.shape)


# speedup vs baseline: 1.4037x; 1.4037x over previous
import jax
import jax.numpy as jnp
from jax.experimental import pallas as pl
from jax.experimental.pallas import tpu as pltpu

_B = 32
_C = 384
_N = 196
_NTOT = float(_B * _N)
_EPS = 1e-5


def _fused(x_hbm, wd_ref, bd_ref, g1_ref, be1_ref, wg_ref, bg_ref,
           wu_ref, bu_ref, g2s_ref, be2s_ref, out_hbm,
           y_ref, xbuf, obuf, isem, osem):
    def in_cp(b):
        return pltpu.make_async_copy(x_hbm.at[b], xbuf.at[b % 2],
                                     isem.at[b % 2])

    def out_cp(b):
        return pltpu.make_async_copy(obuf.at[b % 2], out_hbm.at[b],
                                     osem.at[b % 2])

    wd = wd_ref[...]
    bd = bd_ref[...]
    in_cp(0).start()
    s1 = jnp.zeros((_C, 1), jnp.float32)
    q1 = jnp.zeros((_C, 1), jnp.float32)
    for b in range(_B):
        if b + 1 < _B:
            in_cp(b + 1).start()
        in_cp(b).wait()
        y1 = jnp.dot(wd, xbuf[b % 2],
                     preferred_element_type=jnp.float32) + bd
        y_ref[b] = y1
        s1 = s1 + jnp.sum(y1, axis=1, keepdims=True)
        q1 = q1 + jnp.sum(y1 * y1, axis=1, keepdims=True)
    mu1 = s1 / _NTOT
    var1 = q1 / _NTOT - mu1 * mu1
    a1 = g1_ref[...] * jax.lax.rsqrt(var1 + _EPS)
    c1 = be1_ref[...] - mu1 * a1

    wu = wu_ref[...]
    wc = jnp.dot(wu, wg_ref[...], preferred_element_type=jnp.float32)
    bc = jnp.dot(wu, bg_ref[...], preferred_element_type=jnp.float32) + bu_ref[...]

    s2 = jnp.zeros((_C, 1), jnp.float32)
    q2 = jnp.zeros((_C, 1), jnp.float32)
    for b in range(_B):
        y2 = y_ref[b] * a1 + c1
        y4 = jnp.dot(wc, y2, preferred_element_type=jnp.float32) + bc
        y_ref[b] = y4
        s2 = s2 + jnp.sum(y4, axis=1, keepdims=True)
        q2 = q2 + jnp.sum(y4 * y4, axis=1, keepdims=True)
    mu2 = s2 / _NTOT
    var2 = q2 / _NTOT - mu2 * mu2
    a2 = g2s_ref[...] * jax.lax.rsqrt(var2 + _EPS)
    c2 = be2s_ref[...] - mu2 * a2

    for b in range(_B):
        if b >= 2:
            out_cp(b - 2).wait()
        obuf[b % 2] = y_ref[b] * a2 + c2
        out_cp(b).start()
    out_cp(_B - 2).wait()
    out_cp(_B - 1).wait()


def kernel(x, batch, W_down, b_down, g1, be1, W_gcn, b_gcn, W_up, b_up,
           g2, be2, rel_pos):
    del rel_pos
    scale = jnp.asarray(batch, jnp.float32) / 8.0
    col = lambda v: v.reshape(_C, 1).astype(jnp.float32)
    xr = x.reshape(_B, _C, _N)

    hbm = pl.BlockSpec(memory_space=pltpu.MemorySpace.HBM)
    out = pl.pallas_call(
        _fused,
        in_specs=[hbm] + [pl.BlockSpec(memory_space=pltpu.MemorySpace.VMEM)] * 10,
        out_specs=hbm,
        out_shape=jax.ShapeDtypeStruct((_B, _C, _N), jnp.float32),
        scratch_shapes=[
            pltpu.VMEM((_B, _C, _N), jnp.float32),
            pltpu.VMEM((2, _C, _N), jnp.float32),
            pltpu.VMEM((2, _C, _N), jnp.float32),
            pltpu.SemaphoreType.DMA((2,)),
            pltpu.SemaphoreType.DMA((2,)),
        ],
    )(xr, W_down, col(b_down), col(g1), col(be1), W_gcn, col(b_gcn),
      W_up, col(b_up), col(g2 * scale), col(be2 * scale))
    return out.reshape(x.shape)


# speedup vs baseline: 1.8705x; 1.3326x over previous
import jax
import jax.numpy as jnp
from jax.experimental import pallas as pl

_B = 32
_C = 384
_N = 196
_NTOT = float(_B * _N)
_EPS = 1e-5
_BF = jnp.bfloat16


def _fused(x_ref, wd_ref, bd_ref, g1_ref, be1_ref, wg_ref, bg_ref,
           wu_ref, bu_ref, g2s_ref, be2s_ref, out_ref):
    wd = wd_ref[...].astype(_BF)
    bd = bd_ref[...]
    s1 = jnp.zeros((_C, 1), jnp.float32)
    q1 = jnp.zeros((_C, 1), jnp.float32)
    for b in range(_B):
        y1 = jnp.dot(wd, x_ref[b].astype(_BF),
                     preferred_element_type=jnp.float32) + bd
        out_ref[b] = y1
        s1 = s1 + jnp.sum(y1, axis=1, keepdims=True)
        q1 = q1 + jnp.sum(y1 * y1, axis=1, keepdims=True)
    mu1 = s1 / _NTOT
    var1 = q1 / _NTOT - mu1 * mu1
    a1 = g1_ref[...] * jax.lax.rsqrt(var1 + _EPS)
    c1 = be1_ref[...] - mu1 * a1

    wu = wu_ref[...]
    wc = jnp.dot(wu.astype(_BF), wg_ref[...].astype(_BF),
                 preferred_element_type=jnp.float32).astype(_BF)
    bc = jnp.dot(wu, bg_ref[...], preferred_element_type=jnp.float32) + bu_ref[...]

    s2 = jnp.zeros((_C, 1), jnp.float32)
    q2 = jnp.zeros((_C, 1), jnp.float32)
    for b in range(_B):
        y2 = (out_ref[b] * a1 + c1).astype(_BF)
        y4 = jnp.dot(wc, y2, preferred_element_type=jnp.float32) + bc
        out_ref[b] = y4
        s2 = s2 + jnp.sum(y4, axis=1, keepdims=True)
        q2 = q2 + jnp.sum(y4 * y4, axis=1, keepdims=True)
    mu2 = s2 / _NTOT
    var2 = q2 / _NTOT - mu2 * mu2
    a2 = g2s_ref[...] * jax.lax.rsqrt(var2 + _EPS)
    c2 = be2s_ref[...] - mu2 * a2

    for b in range(_B):
        out_ref[b] = out_ref[b] * a2 + c2


def kernel(x, batch, W_down, b_down, g1, be1, W_gcn, b_gcn, W_up, b_up,
           g2, be2, rel_pos):
    del rel_pos
    scale = jnp.asarray(batch, jnp.float32) / 8.0
    col = lambda v: v.reshape(_C, 1).astype(jnp.float32)
    xr = x.reshape(_B, _C, _N)
    out = pl.pallas_call(
        _fused,
        out_shape=jax.ShapeDtypeStruct((_B, _C, _N), jnp.float32),
    )(xr, W_down, col(b_down), col(g1), col(be1), W_gcn, col(b_gcn),
      W_up, col(b_up), col(g2 * scale), col(be2 * scale))
    return out.reshape(x.shape)
